# Initial kernel scaffold; baseline (speedup 1.0000x reference)
import jax
import jax.numpy as jnp
from jax.experimental import pallas as pl


def kernel(x_author, x_paper, x_conference, edge_index_writes, edge_index_rev_writes, edge_index_published_in, edge_index_rev_published_in, W1_l_writes, W1_r_writes, b1_writes, W2_l_writes, W2_r_writes, b2_writes, W1_l_rev_writes, W1_r_rev_writes, b1_rev_writes, W2_l_rev_writes, W2_r_rev_writes, b2_rev_writes, W1_l_published_in, W1_r_published_in, b1_published_in, W2_l_published_in, W2_r_published_in, b2_published_in, W1_l_rev_published_in, W1_r_rev_published_in, b1_rev_published_in, W2_l_rev_published_in, W2_r_rev_published_in, b2_rev_published_in, W_lin, b_lin):
    raise NotImplementedError("write your pallas kernel here")


# speedup vs baseline: 1.3686x; 1.3686x over previous
import functools

import jax
import jax.numpy as jnp
from jax import lax
from jax.experimental import pallas as pl
from jax.experimental.pallas import tpu as pltpu
from jax.experimental.pallas import tpu_sc as plsc

NW = 32
GB = 128


def _pad_up(n, m):
    return (n + m - 1) // m * m


@functools.lru_cache(maxsize=None)
def _make_seg_sum(D, Ep, NdP, ones_mode):
    G = Ep // NW // GB
    RPT = NdP // 16
    mesh = plsc.VectorSubcoreMesh(core_axis_name="c", subcore_axis_name="s")

    @functools.partial(
        pl.kernel,
        mesh=mesh,
        compiler_params=pltpu.CompilerParams(use_tc_tiling_on_sc=False),
        out_type=jax.ShapeDtypeStruct((2, NdP, D), jnp.float32),
        scratch_types=[
            pltpu.VMEM((G, GB), jnp.int32),
            pltpu.VMEM((G, GB), jnp.int32),
            pltpu.VMEM((GB, D), jnp.float32),
            pltpu.VMEM_SHARED((NdP, D), jnp.float32),
            pltpu.SemaphoreType.DMA,
        ],
    )
    def k(feat_hbm, src_hbm, dst_hbm, zrow_hbm, out_hbm,
          src_v, dst_v, rows_v, acc_sh, sem):
        c = lax.axis_index("c")
        s = lax.axis_index("s")
        wid = s * 2 + c
        pltpu.sync_copy(zrow_hbm, acc_sh.at[pl.ds(s * RPT, RPT)])
        plsc.subcore_barrier()
        pltpu.sync_copy(src_hbm.at[pl.ds(wid * G, G)], src_v)
        pltpu.sync_copy(dst_hbm.at[pl.ds(wid * G, G)], dst_v)
        if ones_mode:
            pltpu.sync_copy(feat_hbm, rows_v)

            def body(g, carry):
                pltpu.sync_copy(rows_v, acc_sh.at[dst_v.at[g]], add=True)
                return carry
        else:
            def body(g, carry):
                pltpu.async_copy(feat_hbm.at[src_v.at[g]], rows_v, sem).wait()
                pltpu.sync_copy(rows_v, acc_sh.at[dst_v.at[g]], add=True)
                return carry

        lax.fori_loop(0, G, body, 0)
        plsc.subcore_barrier()
        pltpu.sync_copy(acc_sh.at[pl.ds(s * RPT, RPT)],
                        out_hbm.at[c].at[pl.ds(s * RPT, RPT)])

    return k


def _pad_edges(src, dst, trash_dst):
    E = src.shape[0]
    Ep = _pad_up(E, NW * GB * 8)
    pad = Ep - E
    src = jnp.concatenate([src, jnp.zeros((pad,), jnp.int32)])
    dst = jnp.concatenate([dst, jnp.full((pad,), trash_dst, jnp.int32)])
    return src.reshape(Ep // GB, GB), dst.reshape(Ep // GB, GB), Ep


def _seg_sum(feat, src2d, dst2d, Ep, NdP):
    D = feat.shape[1]
    zrow = jnp.zeros((NdP // 16, D), jnp.float32)
    out = _make_seg_sum(D, Ep, NdP, False)(feat, src2d, dst2d, zrow)
    return out[0] + out[1]


def _seg_count(dst2d, Ep, NdP):
    ones = jnp.ones((GB, 8), jnp.float32)
    zrow = jnp.zeros((NdP // 16, 8), jnp.float32)
    out = _make_seg_sum(8, Ep, NdP, True)(ones, dst2d, dst2d, zrow)
    return out[0] + out[1]


_BM = 512


def _mm_body(x_ref, w_ref, o_ref):
    o_ref[...] = jnp.dot(x_ref[...], w_ref[...],
                         preferred_element_type=jnp.float32)


def _matmul(x, w):
    M, K = x.shape
    N = w.shape[1]
    grid = (_pad_up(M, _BM) // _BM,)
    return pl.pallas_call(
        _mm_body,
        grid=grid,
        in_specs=[pl.BlockSpec((_BM, K), lambda i: (i, 0)),
                  pl.BlockSpec((K, N), lambda i: (0, 0))],
        out_specs=pl.BlockSpec((_BM, N), lambda i: (i, 0)),
        out_shape=jax.ShapeDtypeStruct((M, N), jnp.float32),
    )(x, w)


def _leaky(h):
    return jnp.where(h > 0, h, 0.01 * h)


def _k2_body(sw_ref, cw_ref, sr_ref, cr_ref, rp_ref, b_ref, w_ref, o_ref):
    cw = jnp.maximum(cw_ref[:, 0:1], 1.0)
    cr = jnp.maximum(cr_ref[:, 0:1], 1.0)
    h = sw_ref[...] / cw + sr_ref[...] / cr + rp_ref[...] + b_ref[...]
    o_ref[...] = jnp.dot(_leaky(h), w_ref[...],
                         preferred_element_type=jnp.float32)


def _k2(sum_w, cnt_w, sum_r, cnt_r, r_paper, bias_p, w2):
    M = sum_w.shape[0]
    grid = (_pad_up(M, _BM) // _BM,)
    row = pl.BlockSpec((_BM, 32), lambda i: (i, 0))
    cntspec = pl.BlockSpec((_BM, 8), lambda i: (i, 0))
    return pl.pallas_call(
        _k2_body,
        grid=grid,
        in_specs=[row, cntspec, row, cntspec, row,
                  pl.BlockSpec((1, 32), lambda i: (0, 0)),
                  pl.BlockSpec((32, 32), lambda i: (0, 0))],
        out_specs=row,
        out_shape=jax.ShapeDtypeStruct((M, 32), jnp.float32),
    )(sum_w, cnt_w, sum_r, cnt_r, r_paper, bias_p.reshape(1, 32), w2)


def _k3_body(s1_ref, cnt_ref, rc_ref, b1_ref, s2_ref, b2_ref,
             wr_ref, wl_ref, bl_ref, o_ref):
    cnt = jnp.maximum(cnt_ref[:, 0:1], 1.0)
    h1 = _leaky(s1_ref[...] / cnt + rc_ref[...] + b1_ref[...])
    h2 = _leaky(s2_ref[...] / cnt + b2_ref[...] +
                jnp.dot(h1, wr_ref[...], preferred_element_type=jnp.float32))
    o_ref[...] = jnp.dot(h2, wl_ref[...],
                         preferred_element_type=jnp.float32) + bl_ref[...]


def _k3(sum1, cnt, r_conf, b1, sum2, b2, w2r, wlin_pad, blin_pad):
    M = sum1.shape[0]
    grid = (_pad_up(M, _BM) // _BM,)
    row = pl.BlockSpec((_BM, 32), lambda i: (i, 0))
    return pl.pallas_call(
        _k3_body,
        grid=grid,
        in_specs=[row, pl.BlockSpec((_BM, 8), lambda i: (i, 0)), row,
                  pl.BlockSpec((1, 32), lambda i: (0, 0)), row,
                  pl.BlockSpec((1, 32), lambda i: (0, 0)),
                  pl.BlockSpec((32, 32), lambda i: (0, 0)),
                  pl.BlockSpec((32, 8), lambda i: (0, 0)),
                  pl.BlockSpec((1, 8), lambda i: (0, 0))],
        out_specs=pl.BlockSpec((_BM, 8), lambda i: (i, 0)),
        out_shape=jax.ShapeDtypeStruct((M, 8), jnp.float32),
    )(sum1, cnt, r_conf, b1.reshape(1, 32), sum2, b2.reshape(1, 32),
      w2r, wlin_pad, blin_pad.reshape(1, 8))


N_AUTHOR, N_PAPER, N_CONF = 50000, 100000, 10000
NDP_P = _pad_up(N_PAPER + 1, 128)
NDP_C = _pad_up(N_CONF + 1, 128)


def kernel(x_author, x_paper, x_conference,
           edge_index_writes, edge_index_rev_writes,
           edge_index_published_in, edge_index_rev_published_in,
           W1_l_writes, W1_r_writes, b1_writes,
           W2_l_writes, W2_r_writes, b2_writes,
           W1_l_rev_writes, W1_r_rev_writes, b1_rev_writes,
           W2_l_rev_writes, W2_r_rev_writes, b2_rev_writes,
           W1_l_published_in, W1_r_published_in, b1_published_in,
           W2_l_published_in, W2_r_published_in, b2_published_in,
           W1_l_rev_published_in, W1_r_rev_published_in, b1_rev_published_in,
           W2_l_rev_published_in, W2_r_rev_published_in, b2_rev_published_in,
           W_lin, b_lin):
    p_writes = _matmul(x_author, W1_l_writes)
    y_conf = _matmul(x_conference,
                     jnp.concatenate([W1_l_rev_published_in,
                                      W1_r_published_in], axis=1))
    p_rpi, r_conf = y_conf[:, :32], y_conf[:, 32:]
    y_paper = _matmul(x_paper,
                      jnp.concatenate([W1_l_published_in,
                                       W1_r_writes + W1_r_rev_published_in],
                                      axis=1))
    p_pi, r_paper = y_paper[:, :32], y_paper[:, 32:]

    sw2, dw2, ep_w = _pad_edges(edge_index_writes[0],
                                edge_index_writes[1], N_PAPER)
    sr2, dr2, ep_r = _pad_edges(edge_index_rev_published_in[0],
                                edge_index_rev_published_in[1], N_PAPER)
    sp2, dp2, ep_p = _pad_edges(edge_index_published_in[0],
                                edge_index_published_in[1], N_CONF)

    cnt_w = _seg_count(dw2, ep_w, NDP_P)
    dall = jnp.concatenate([dr2.reshape(-1), dp2.reshape(-1) + NDP_P])
    cnts = _seg_count(dall.reshape(-1, GB), ep_r + ep_p, NDP_P + NDP_C)
    cnt_r = cnts[:NDP_P]
    cnt_p = cnts[NDP_P:]

    sum_w = jnp.concatenate(
        [_seg_sum(p_writes[:, q * 8:(q + 1) * 8], sw2, dw2, ep_w, NDP_P)
         for q in range(4)], axis=1)
    sum_r = jnp.concatenate(
        [_seg_sum(p_rpi[:, q * 8:(q + 1) * 8], sr2, dr2, ep_r, NDP_P)
         for q in range(4)], axis=1)
    sum_p1 = _seg_sum(p_pi, sp2, dp2, ep_p, NDP_C)

    r_paper_pad = jnp.pad(r_paper, ((0, NDP_P - N_PAPER), (0, 0)))
    feat2 = _k2(sum_w, cnt_w, sum_r, cnt_r, r_paper_pad,
                b1_writes + b1_rev_published_in, W2_l_published_in)

    sum_p2 = _seg_sum(feat2, sp2, dp2, ep_p, NDP_C)

    r_conf_pad = jnp.pad(r_conf, ((0, NDP_C - N_CONF), (0, 0)))
    wlin_pad = jnp.pad(W_lin, ((0, 0), (0, 6)))
    blin_pad = jnp.pad(b_lin, (0, 6))
    out = _k3(sum_p1, cnt_p, r_conf_pad, b1_published_in,
              sum_p2, b2_published_in, W2_r_published_in,
              wlin_pad, blin_pad)
    return out[:N_CONF, :2]


# speedup vs baseline: 1.3855x; 1.0123x over previous
import functools

import jax
import jax.numpy as jnp
from jax import lax
from jax.experimental import pallas as pl
from jax.experimental.pallas import tpu as pltpu
from jax.experimental.pallas import tpu_sc as plsc

NW = 32
GB = 128


def _pad_up(n, m):
    return (n + m - 1) // m * m


@functools.lru_cache(maxsize=None)
def _make_seg_sum(D, Ep, NdP, ones_mode):
    G = Ep // NW // GB
    RPT = NdP // 16
    mesh = plsc.VectorSubcoreMesh(core_axis_name="c", subcore_axis_name="s")

    @functools.partial(
        pl.kernel,
        mesh=mesh,
        compiler_params=pltpu.CompilerParams(use_tc_tiling_on_sc=False),
        out_type=jax.ShapeDtypeStruct((2, NdP, D), jnp.float32),
        scratch_types=[
            pltpu.VMEM((G, GB), jnp.int32),
            pltpu.VMEM((G, GB), jnp.int32),
            pltpu.VMEM((2, GB, D), jnp.float32),
            pltpu.VMEM_SHARED((NdP, D), jnp.float32),
            pltpu.SemaphoreType.DMA,
        ],
    )
    def k(feat_hbm, src_hbm, dst_hbm, zrow_hbm, out_hbm,
          src_v, dst_v, rows_v, acc_sh, sem):
        c = lax.axis_index("c")
        s = lax.axis_index("s")
        wid = s * 2 + c
        pltpu.sync_copy(zrow_hbm, acc_sh.at[pl.ds(s * RPT, RPT)])
        plsc.subcore_barrier()
        pltpu.sync_copy(src_hbm.at[pl.ds(wid * G, G)], src_v)
        pltpu.sync_copy(dst_hbm.at[pl.ds(wid * G, G)], dst_v)
        if ones_mode:
            pltpu.sync_copy(feat_hbm, rows_v.at[0])

            def body(g, carry):
                pltpu.sync_copy(rows_v.at[0], acc_sh.at[dst_v.at[g]],
                                add=True)
                return carry
        else:
            pltpu.async_copy(feat_hbm.at[src_v.at[0]], rows_v.at[0], sem)

            def body(g, carry):
                @pl.when(g + 1 < G)
                def _():
                    pltpu.async_copy(feat_hbm.at[src_v.at[g + 1]],
                                     rows_v.at[(g + 1) % 2], sem)
                pltpu.make_async_copy(feat_hbm.at[src_v.at[g]],
                                      rows_v.at[g % 2], sem).wait()
                pltpu.sync_copy(rows_v.at[g % 2], acc_sh.at[dst_v.at[g]],
                                add=True)
                return carry

        lax.fori_loop(0, G, body, 0)
        plsc.subcore_barrier()
        pltpu.sync_copy(acc_sh.at[pl.ds(s * RPT, RPT)],
                        out_hbm.at[c].at[pl.ds(s * RPT, RPT)])

    return k


def _pad_edges(src, dst, trash_dst):
    E = src.shape[0]
    Ep = _pad_up(E, NW * GB * 8)
    pad = Ep - E
    src = jnp.concatenate([src, jnp.zeros((pad,), jnp.int32)])
    dst = jnp.concatenate([dst, jnp.full((pad,), trash_dst, jnp.int32)])
    return src.reshape(Ep // GB, GB), dst.reshape(Ep // GB, GB), Ep


def _seg_sum(feat, src2d, dst2d, Ep, NdP):
    D = feat.shape[1]
    zrow = jnp.zeros((NdP // 16, D), jnp.float32)
    out = _make_seg_sum(D, Ep, NdP, False)(feat, src2d, dst2d, zrow)
    return out[0] + out[1]


def _seg_count(dst2d, Ep, NdP):
    ones = jnp.ones((GB, 8), jnp.float32)
    zrow = jnp.zeros((NdP // 16, 8), jnp.float32)
    out = _make_seg_sum(8, Ep, NdP, True)(ones, dst2d, dst2d, zrow)
    return out[0] + out[1]


_BM = 512


def _mm_body(x_ref, w_ref, o_ref):
    o_ref[...] = jnp.dot(x_ref[...], w_ref[...],
                         preferred_element_type=jnp.float32)


def _matmul(x, w):
    M, K = x.shape
    N = w.shape[1]
    grid = (_pad_up(M, _BM) // _BM,)
    return pl.pallas_call(
        _mm_body,
        grid=grid,
        in_specs=[pl.BlockSpec((_BM, K), lambda i: (i, 0)),
                  pl.BlockSpec((K, N), lambda i: (0, 0))],
        out_specs=pl.BlockSpec((_BM, N), lambda i: (i, 0)),
        out_shape=jax.ShapeDtypeStruct((M, N), jnp.float32),
    )(x, w)


def _leaky(h):
    return jnp.where(h > 0, h, 0.01 * h)


def _k2_body(sw_ref, cw_ref, sr_ref, cr_ref, rp_ref, b_ref, w_ref, o_ref):
    cw = jnp.maximum(cw_ref[:, 0:1], 1.0)
    cr = jnp.maximum(cr_ref[:, 0:1], 1.0)
    h = sw_ref[...] / cw + sr_ref[...] / cr + rp_ref[...] + b_ref[...]
    o_ref[...] = jnp.dot(_leaky(h), w_ref[...],
                         preferred_element_type=jnp.float32)


def _k2(sum_w, cnt_w, sum_r, cnt_r, r_paper, bias_p, w2):
    M = sum_w.shape[0]
    grid = (_pad_up(M, _BM) // _BM,)
    row = pl.BlockSpec((_BM, 32), lambda i: (i, 0))
    cntspec = pl.BlockSpec((_BM, 8), lambda i: (i, 0))
    return pl.pallas_call(
        _k2_body,
        grid=grid,
        in_specs=[row, cntspec, row, cntspec, row,
                  pl.BlockSpec((1, 32), lambda i: (0, 0)),
                  pl.BlockSpec((32, 32), lambda i: (0, 0))],
        out_specs=row,
        out_shape=jax.ShapeDtypeStruct((M, 32), jnp.float32),
    )(sum_w, cnt_w, sum_r, cnt_r, r_paper, bias_p.reshape(1, 32), w2)


def _k3_body(s1_ref, cnt_ref, rc_ref, b1_ref, s2_ref, b2_ref,
             wr_ref, wl_ref, bl_ref, o_ref):
    cnt = jnp.maximum(cnt_ref[:, 0:1], 1.0)
    h1 = _leaky(s1_ref[...] / cnt + rc_ref[...] + b1_ref[...])
    h2 = _leaky(s2_ref[...] / cnt + b2_ref[...] +
                jnp.dot(h1, wr_ref[...], preferred_element_type=jnp.float32))
    o_ref[...] = jnp.dot(h2, wl_ref[...],
                         preferred_element_type=jnp.float32) + bl_ref[...]


def _k3(sum1, cnt, r_conf, b1, sum2, b2, w2r, wlin_pad, blin_pad):
    M = sum1.shape[0]
    grid = (_pad_up(M, _BM) // _BM,)
    row = pl.BlockSpec((_BM, 32), lambda i: (i, 0))
    return pl.pallas_call(
        _k3_body,
        grid=grid,
        in_specs=[row, pl.BlockSpec((_BM, 8), lambda i: (i, 0)), row,
                  pl.BlockSpec((1, 32), lambda i: (0, 0)), row,
                  pl.BlockSpec((1, 32), lambda i: (0, 0)),
                  pl.BlockSpec((32, 32), lambda i: (0, 0)),
                  pl.BlockSpec((32, 8), lambda i: (0, 0)),
                  pl.BlockSpec((1, 8), lambda i: (0, 0))],
        out_specs=pl.BlockSpec((_BM, 8), lambda i: (i, 0)),
        out_shape=jax.ShapeDtypeStruct((M, 8), jnp.float32),
    )(sum1, cnt, r_conf, b1.reshape(1, 32), sum2, b2.reshape(1, 32),
      w2r, wlin_pad, blin_pad.reshape(1, 8))


N_AUTHOR, N_PAPER, N_CONF = 50000, 100000, 10000
NDP_P = _pad_up(N_PAPER + 1, 128)
NDP_C = _pad_up(N_CONF + 1, 128)


def kernel(x_author, x_paper, x_conference,
           edge_index_writes, edge_index_rev_writes,
           edge_index_published_in, edge_index_rev_published_in,
           W1_l_writes, W1_r_writes, b1_writes,
           W2_l_writes, W2_r_writes, b2_writes,
           W1_l_rev_writes, W1_r_rev_writes, b1_rev_writes,
           W2_l_rev_writes, W2_r_rev_writes, b2_rev_writes,
           W1_l_published_in, W1_r_published_in, b1_published_in,
           W2_l_published_in, W2_r_published_in, b2_published_in,
           W1_l_rev_published_in, W1_r_rev_published_in, b1_rev_published_in,
           W2_l_rev_published_in, W2_r_rev_published_in, b2_rev_published_in,
           W_lin, b_lin):
    p_writes = _matmul(x_author, W1_l_writes)
    y_conf = _matmul(x_conference,
                     jnp.concatenate([W1_l_rev_published_in,
                                      W1_r_published_in], axis=1))
    p_rpi, r_conf = y_conf[:, :32], y_conf[:, 32:]
    y_paper = _matmul(x_paper,
                      jnp.concatenate([W1_l_published_in,
                                       W1_r_writes + W1_r_rev_published_in],
                                      axis=1))
    p_pi, r_paper = y_paper[:, :32], y_paper[:, 32:]

    sw2, dw2, ep_w = _pad_edges(edge_index_writes[0],
                                edge_index_writes[1], N_PAPER)
    sr2, dr2, ep_r = _pad_edges(edge_index_rev_published_in[0],
                                edge_index_rev_published_in[1], N_PAPER)
    sp2, dp2, ep_p = _pad_edges(edge_index_published_in[0],
                                edge_index_published_in[1], N_CONF)

    cnt_w = _seg_count(dw2, ep_w, NDP_P)
    dall = jnp.concatenate([dr2.reshape(-1), dp2.reshape(-1) + NDP_P])
    cnts = _seg_count(dall.reshape(-1, GB), ep_r + ep_p, NDP_P + NDP_C)
    cnt_r = cnts[:NDP_P]
    cnt_p = cnts[NDP_P:]

    sum_w = jnp.concatenate(
        [_seg_sum(p_writes[:, q * 8:(q + 1) * 8], sw2, dw2, ep_w, NDP_P)
         for q in range(4)], axis=1)
    sum_r = jnp.concatenate(
        [_seg_sum(p_rpi[:, q * 8:(q + 1) * 8], sr2, dr2, ep_r, NDP_P)
         for q in range(4)], axis=1)
    sum_p1 = _seg_sum(p_pi, sp2, dp2, ep_p, NDP_C)

    r_paper_pad = jnp.pad(r_paper, ((0, NDP_P - N_PAPER), (0, 0)))
    feat2 = _k2(sum_w, cnt_w, sum_r, cnt_r, r_paper_pad,
                b1_writes + b1_rev_published_in, W2_l_published_in)

    sum_p2 = _seg_sum(feat2, sp2, dp2, ep_p, NDP_C)

    r_conf_pad = jnp.pad(r_conf, ((0, NDP_C - N_CONF), (0, 0)))
    wlin_pad = jnp.pad(W_lin, ((0, 0), (0, 6)))
    blin_pad = jnp.pad(b_lin, (0, 6))
    out = _k3(sum_p1, cnt_p, r_conf_pad, b1_published_in,
              sum_p2, b2_published_in, W2_r_published_in,
              wlin_pad, blin_pad)
    return out[:N_CONF, :2]
